# Initial kernel scaffold; baseline (speedup 1.0000x reference)
import jax
import jax.numpy as jnp
from jax.experimental import pallas as pl


def kernel(x, input_pts, params):
    raise NotImplementedError("write your pallas kernel here")


# speedup vs baseline: 6.8365x; 6.8365x over previous
import functools

import jax
import jax.numpy as jnp
from jax import lax
from jax.experimental import pallas as pl
from jax.experimental.pallas import tpu as pltpu
from jax.experimental.pallas import tpu_sc as plsc

_DIM = 3
_NC = 16
_HI = jax.lax.Precision.HIGHEST
_H3 = jax.lax.Precision.HIGHEST

_INTERPRET = False


def _layer_body(q_ref, pts_ref, ptst_ref, x_ref, c_ref, l1wt_ref, l1b_ref,
                l2wt_ref, l2b_ref, l3wt_ref, l3b_ref, w2_ref, o_ref,
                gp_s, feats_s, *, K, Nn, bn, cin):
    q = q_ref[0]
    p = pts_ref[0]
    xx = x_ref[0]

    d = None
    for dd in range(_DIM):
        diff = q[:, dd:dd + 1] - ptst_ref[0, dd:dd + 1, :]
        t = diff * diff
        d = t if d is None else d + t

    iota = lax.broadcasted_iota(jnp.int32, (bn, Nn), 1).astype(jnp.float32)
    big = jnp.float32(1e30)

    def topk_step(k, dc):
        m = jnp.min(dc, axis=1, keepdims=True)
        cand = jnp.where(dc == m, iota, jnp.float32(Nn))
        am = jnp.min(cand, axis=1, keepdims=True)
        onehot = (iota == am).astype(jnp.float32)
        gp_s[pl.ds(k * bn, bn), :] = jnp.dot(onehot, p, precision=_H3)
        feats_s[pl.ds(k * bn, bn), :] = jnp.dot(onehot, xx, precision=_H3)
        return dc + onehot * big

    lax.fori_loop(0, K, topk_step, d, unroll=False)

    feats = [feats_s[k * bn:(k + 1) * bn, :] for k in range(K)]
    gp = [gp_s[k * bn:(k + 1) * bn, :] for k in range(K)]

    o_ref[0] = _aggregate(q, gp, feats, c_ref, l1wt_ref, l1b_ref, l2wt_ref,
                          l2b_ref, l3wt_ref, l3b_ref, w2_ref, K=K, bn=bn)


def _aggregate(q, gp, feats, c_ref, l1wt_ref, l1b_ref, l2wt_ref, l2b_ref,
               l3wt_ref, l3b_ref, w2_ref, *, K, bn):
    gk = [g - q for g in gp]
    r2 = [g[:, 0:1] * g[:, 0:1] + g[:, 1:2] * g[:, 1:2] + g[:, 2:3] * g[:, 2:3]
          for g in gk]
    maxr = r2[0]
    for r in r2[1:]:
        maxr = jnp.maximum(maxr, r)
    maxi = jnp.sqrt(maxr)
    maxi = jnp.where(maxi == 0.0, jnp.float32(1.0), maxi)

    dcat = []
    for k in range(K):
        gn = gk[k] / maxi
        parts = [gn[:, dd:dd + 1] - c_ref[dd:dd + 1, :] for dd in range(_DIM)]
        dcat.append(jnp.concatenate(parts, axis=1))
    dmat = jnp.concatenate(dcat, axis=0)

    bf = jnp.bfloat16
    h = jnp.maximum(jnp.dot(dmat.astype(bf), l1wt_ref[...],
                            preferred_element_type=jnp.float32) + l1b_ref[...], 0.0)
    h = jnp.maximum(jnp.dot(h.astype(bf), l2wt_ref[...],
                            preferred_element_type=jnp.float32) + l2b_ref[...], 0.0)
    h = jnp.maximum(jnp.dot(h.astype(bf), l3wt_ref[...],
                            preferred_element_type=jnp.float32) + l3b_ref[...], 0.0)

    fb = [feats[k].astype(bf).astype(jnp.float32) for k in range(K)]
    hb = h.astype(bf).astype(jnp.float32)
    facc = []
    for mm in range(_NC):
        a = None
        for k in range(K):
            t = hb[k * bn:(k + 1) * bn, mm:mm + 1] * fb[k]
            a = t if a is None else a + t
        facc.append(a)
    fmat = jnp.concatenate(facc, axis=1)

    return jnp.dot(fmat.astype(bf), w2_ref[...],
                   preferred_element_type=jnp.float32) * jnp.float32(1.0 / K)


def _topk_body(q_ref, ptst_ref, idx_ref, *, K, Nn, bn):
    q = q_ref[0]
    d = None
    for dd in range(_DIM):
        diff = q[:, dd:dd + 1] - ptst_ref[0, dd:dd + 1, :]
        t = diff * diff
        d = t if d is None else d + t

    iota = lax.broadcasted_iota(jnp.int32, (bn, Nn), 1).astype(jnp.float32)
    kiota = lax.broadcasted_iota(jnp.int32, (bn, K), 1)
    big = jnp.float32(1e30)

    def step(k, carry):
        dc, idxacc = carry
        m = jnp.min(dc, axis=1, keepdims=True)
        cand = jnp.where(dc == m, iota, jnp.float32(Nn))
        am = jnp.min(cand, axis=1, keepdims=True)
        idxacc = jnp.where(kiota == k, am.astype(jnp.int32), idxacc)
        dc = jnp.where(iota == am, big, dc)
        return dc, idxacc

    _, idxacc = lax.fori_loop(0, K, step,
                              (d, jnp.zeros((bn, K), jnp.int32)))
    idx_ref[0] = idxacc + pl.program_id(0) * Nn


def _sc_gather(table, idx, chunk):
    R = idx.shape[0]
    D = table.shape[1]
    nw = 32
    per_w = R // nw
    nch = per_w // chunk
    mesh = plsc.VectorSubcoreMesh(core_axis_name="c", subcore_axis_name="s")

    @functools.partial(
        pl.kernel, mesh=mesh,
        out_type=jax.ShapeDtypeStruct((R, D), jnp.float32),
        scratch_types=[
            pltpu.VMEM((chunk,), jnp.int32),
            pltpu.VMEM((chunk, D), jnp.float32),
            pltpu.SemaphoreType.DMA,
        ],
    )
    def g(table_hbm, idx_hbm, out_hbm, idx_v, rows_v, sem):
        wid = lax.axis_index("s") * 2 + lax.axis_index("c")
        for ci in range(nch):
            base = wid * per_w + ci * chunk
            pltpu.sync_copy(idx_hbm.at[pl.ds(base, chunk)], idx_v)
            pltpu.async_copy(table_hbm.at[idx_v], rows_v, sem).wait()
            pltpu.sync_copy(rows_v, out_hbm.at[pl.ds(base, chunk)])

    return g(table, idx)


def _agg_body(q_ref, comb_ref, c_ref, l1wt_ref, l1b_ref, l2wt_ref, l2b_ref,
              l3wt_ref, l3b_ref, w2_ref, o_ref, *, K, bn, cin):
    q = q_ref[0]
    gp = [comb_ref[0, k][:, 0:_DIM] for k in range(K)]
    feats = [comb_ref[0, k][:, _NC:_NC + cin] for k in range(K)]
    o_ref[0] = _aggregate(q, gp, feats, c_ref, l1wt_ref, l1b_ref, l2wt_ref,
                          l2b_ref, l3wt_ref, l3b_ref, w2_ref, K=K, bn=bn)


def _ptconv_layer_sc(x, pts, ptst, prm, K, n_next, bn, chunk):
    B, Nn, cin = x.shape
    cout = prm['w'].shape[1]
    w2 = prm['w'].reshape(cin, _NC, cout).transpose(1, 0, 2).reshape(_NC * cin, cout)
    grid = (B, n_next // bn)

    idx = pl.pallas_call(
        functools.partial(_topk_body, K=K, Nn=Nn, bn=bn),
        grid=grid,
        in_specs=[
            pl.BlockSpec((1, bn, _DIM), lambda b, i: (b, i, 0)),
            pl.BlockSpec((1, _DIM, Nn), lambda b, i: (b, 0, 0)),
        ],
        out_specs=pl.BlockSpec((1, bn, K), lambda b, i: (b, i, 0)),
        out_shape=jax.ShapeDtypeStruct((B, n_next, K), jnp.int32),
        interpret=_INTERPRET,
    )(pts, ptst)

    idx_flat = jnp.transpose(idx, (0, 2, 1)).reshape(-1)
    D = 128
    table = jnp.concatenate([
        jnp.pad(pts.reshape(B * Nn, _DIM), ((0, 0), (0, _NC - _DIM))),
        jnp.pad(x.reshape(B * Nn, cin), ((0, 0), (0, D - _NC - cin)))],
        axis=1)
    if _INTERPRET:
        comb = table[idx_flat]
    else:
        comb = _sc_gather(table, idx_flat, chunk)
    comb = comb.reshape(B, K, n_next, D)

    out = pl.pallas_call(
        functools.partial(_agg_body, K=K, bn=bn, cin=cin),
        grid=grid,
        in_specs=[
            pl.BlockSpec((1, bn, _DIM), lambda b, i: (b, i, 0)),
            pl.BlockSpec((1, K, bn, D), lambda b, i: (b, 0, i, 0)),
            pl.BlockSpec((_DIM, _NC), lambda b, i: (0, 0)),
            pl.BlockSpec((_DIM * _NC, 2 * _NC), lambda b, i: (0, 0)),
            pl.BlockSpec((1, 2 * _NC), lambda b, i: (0, 0)),
            pl.BlockSpec((2 * _NC, _NC), lambda b, i: (0, 0)),
            pl.BlockSpec((1, _NC), lambda b, i: (0, 0)),
            pl.BlockSpec((_NC, _NC), lambda b, i: (0, 0)),
            pl.BlockSpec((1, _NC), lambda b, i: (0, 0)),
            pl.BlockSpec((_NC * cin, cout), lambda b, i: (0, 0)),
        ],
        out_specs=pl.BlockSpec((1, bn, cout), lambda b, i: (b, i, 0)),
        out_shape=jax.ShapeDtypeStruct((B, n_next, cout), jnp.float32),
        interpret=_INTERPRET,
    )(pts, comb, prm['c'],
      prm['l1w'].T.astype(jnp.bfloat16), prm['l1b'][None, :],
      prm['l2w'].T.astype(jnp.bfloat16), prm['l2b'][None, :],
      prm['l3w'].T.astype(jnp.bfloat16), prm['l3b'][None, :],
      w2.astype(jnp.bfloat16))
    return out


def _ptconv_layer(x, pts, ptst, prm, K, n_next, bn):
    B, Nn, cin = x.shape
    cout = prm['w'].shape[1]
    w2 = prm['w'].reshape(cin, _NC, cout).transpose(1, 0, 2).reshape(_NC * cin, cout)
    grid = (B, n_next // bn)
    body = functools.partial(_layer_body, K=K, Nn=Nn, bn=bn, cin=cin)
    out = pl.pallas_call(
        body,
        grid=grid,
        in_specs=[
            pl.BlockSpec((1, bn, _DIM), lambda b, i: (b, i, 0)),
            pl.BlockSpec((1, Nn, _DIM), lambda b, i: (b, 0, 0)),
            pl.BlockSpec((1, _DIM, Nn), lambda b, i: (b, 0, 0)),
            pl.BlockSpec((1, Nn, cin), lambda b, i: (b, 0, 0)),
            pl.BlockSpec((_DIM, _NC), lambda b, i: (0, 0)),
            pl.BlockSpec((_DIM * _NC, 2 * _NC), lambda b, i: (0, 0)),
            pl.BlockSpec((1, 2 * _NC), lambda b, i: (0, 0)),
            pl.BlockSpec((2 * _NC, _NC), lambda b, i: (0, 0)),
            pl.BlockSpec((1, _NC), lambda b, i: (0, 0)),
            pl.BlockSpec((_NC, _NC), lambda b, i: (0, 0)),
            pl.BlockSpec((1, _NC), lambda b, i: (0, 0)),
            pl.BlockSpec((_NC * cin, cout), lambda b, i: (0, 0)),
        ],
        out_specs=pl.BlockSpec((1, bn, cout), lambda b, i: (b, i, 0)),
        out_shape=jax.ShapeDtypeStruct((B, n_next, cout), jnp.float32),
        scratch_shapes=[
            pltpu.VMEM((K * bn, _DIM), jnp.float32),
            pltpu.VMEM((K * bn, cin), jnp.float32),
        ],
        interpret=_INTERPRET,
    )(pts, pts, ptst, x, prm['c'],
      prm['l1w'].T.astype(jnp.bfloat16), prm['l1b'][None, :],
      prm['l2w'].T.astype(jnp.bfloat16), prm['l2b'][None, :],
      prm['l3w'].T.astype(jnp.bfloat16), prm['l3b'][None, :],
      w2.astype(jnp.bfloat16))
    return out


def _bn_relu_body(x_ref, g_ref, b_ref, o_ref):
    xv = x_ref[...]
    m = jnp.mean(xv, axis=0, keepdims=True)
    cen = xv - m
    v = jnp.mean(cen * cen, axis=0, keepdims=True)
    o_ref[...] = jnp.maximum(cen / jnp.sqrt(v + 1e-5) * g_ref[...] + b_ref[...], 0.0)


def _bn_relu(x, gamma, beta):
    B, n, c = x.shape
    flat = x.reshape(B * n, c)
    out = pl.pallas_call(
        _bn_relu_body,
        in_specs=[
            pl.BlockSpec((B * n, c), lambda: (0, 0)),
            pl.BlockSpec((1, c), lambda: (0, 0)),
            pl.BlockSpec((1, c), lambda: (0, 0)),
        ],
        out_specs=pl.BlockSpec((B * n, c), lambda: (0, 0)),
        out_shape=jax.ShapeDtypeStruct((B * n, c), jnp.float32),
        interpret=_INTERPRET,
    )(flat, gamma[None, :], beta[None, :])
    return out.reshape(B, n, c)


_LAYERS = (
    ('cv2', 'bn2', 16, 1024, 128, 512),
    ('cv3', 'bn3', 16, 256, 128, 512),
    ('cv4', 'bn4', 8, 64, 64, 0),
    ('cv5', 'bn5', 8, 16, 16, 0),
    ('cv6', 'bn6', 4, 8, 8, 0),
)


def kernel(x, input_pts, params):
    pts = input_pts
    ptst = jnp.swapaxes(input_pts, 1, 2)
    outs = []
    cur = x
    for cv, bnname, K, n_next, bn, chunk in _LAYERS:
        if chunk:
            y = _ptconv_layer_sc(cur, pts, ptst, params[cv], K, n_next, bn, chunk)
        else:
            y = _ptconv_layer(cur, pts, ptst, params[cv], K, n_next, bn)
        gamma, beta = params[bnname]
        y = _bn_relu(y, gamma, beta)
        pts = pts[:, :n_next]
        ptst = ptst[:, :, :n_next]
        outs.append((y, pts))
        cur = y
    (x2, pts2), (x3, pts3), (x4, pts4), (x5, pts5), (x6, pts6) = outs
    return (x6, pts6, x5, pts5, x4, pts4, x3, pts3, x2, pts2)
